# Initial kernel scaffold; baseline (speedup 1.0000x reference)
import functools

import jax
import jax.numpy as jnp
from jax import lax
from jax.experimental import pallas as pl
from jax.experimental.pallas import tpu as pltpu
from jax.experimental.pallas import tpu_sc as plsc

N = 10000
E = 320000
D = 128
G = 64
EPS = 1e-5

NC = 2
NS = 16
NW = NC * NS
EPT = E // NW
K = 80
NWIN = EPT // K
RPT = N // NS

NBLK = 25
BR = N // NBLK

_mesh = plsc.VectorSubcoreMesh(core_axis_name="c", subcore_axis_name="s")


def _hist_body(row_hbm, zero_hbm, ones_hbm, out_hbm, idx_v, ones_v, hist_s):
    cid = lax.axis_index("c")
    sid = lax.axis_index("s")
    wid = sid * NC + cid

    pltpu.sync_copy(ones_hbm, ones_v)

    @pl.when(sid == 0)
    def _():
        pltpu.sync_copy(zero_hbm, hist_s)

    plsc.subcore_barrier()

    def body(w, carry):
        base = wid * EPT + w * K
        pltpu.sync_copy(row_hbm.at[pl.ds(base, K)], idx_v)
        pltpu.sync_copy(ones_v, hist_s.at[idx_v], add=True)
        return carry

    lax.fori_loop(0, NWIN, body, 0)
    plsc.subcore_barrier()

    @pl.when(sid == 0)
    def _():
        pltpu.sync_copy(hist_s, out_hbm.at[cid])


_hist_call = functools.partial(
    pl.kernel,
    out_type=jax.ShapeDtypeStruct((NC, N, 1), jnp.float32),
    mesh=_mesh,
    scratch_types=[
        pltpu.VMEM((K,), jnp.int32),
        pltpu.VMEM((K, 1), jnp.float32),
        pltpu.VMEM_SHARED((N, 1), jnp.float32),
    ],
)(_hist_body)


def _scat_body(hs_hbm, row_hbm, col_hbm, ea_hbm, zero_hbm, out_hbm,
               ridx_v, cidx_v, ea_v, rows_v, sem, acc_s):
    cid = lax.axis_index("c")
    sid = lax.axis_index("s")
    wid = sid * NC + cid

    pltpu.sync_copy(zero_hbm.at[pl.ds(sid * RPT, RPT)],
                    acc_s.at[pl.ds(sid * RPT, RPT)])
    plsc.subcore_barrier()

    def body(w, carry):
        base = wid * EPT + w * K
        pltpu.sync_copy(row_hbm.at[pl.ds(base, K)], ridx_v)
        pltpu.sync_copy(col_hbm.at[pl.ds(base, K)], cidx_v)
        pltpu.sync_copy(ea_hbm.at[pl.ds(base, K)], ea_v)
        pltpu.async_copy(hs_hbm.at[ridx_v], rows_v, sem).wait()

        def sbody(j, c2):
            s = jnp.zeros((16,), jnp.int32) + j
            sc = plsc.load_gather(ea_v, [s])
            for v in range(D // 16):
                sl = pl.ds(v * 16, 16)
                rows_v[j, sl] = rows_v[j, sl] * sc
            return c2

        lax.fori_loop(0, K, sbody, 0)
        pltpu.sync_copy(rows_v, acc_s.at[cidx_v], add=True)
        return carry

    lax.fori_loop(0, NWIN, body, 0)
    plsc.subcore_barrier()
    pltpu.sync_copy(acc_s.at[pl.ds(sid * RPT, RPT)],
                    out_hbm.at[cid, pl.ds(sid * RPT, RPT)])


_scat_call = functools.partial(
    pl.kernel,
    out_type=jax.ShapeDtypeStruct((NC, N, D), jnp.float32),
    mesh=_mesh,
    scratch_types=[
        pltpu.VMEM((K,), jnp.int32),
        pltpu.VMEM((K,), jnp.int32),
        pltpu.VMEM((K,), jnp.float32),
        pltpu.VMEM((K, D), jnp.float32),
        pltpu.SemaphoreType.DMA,
        pltpu.VMEM_SHARED((N, D), jnp.float32),
    ],
)(_scat_body)


def _pre_body(x_ref, w1_ref, b1_ref, g1_ref, be1_ref, m1_ref, v1_ref,
              w2_ref, b2_ref, hist_ref, hs_ref):
    h = jnp.dot(x_ref[...], w1_ref[...], preferred_element_type=jnp.float32)
    h = h + b1_ref[...]
    s1 = g1_ref[...] * lax.rsqrt(v1_ref[...] + EPS)
    h = (h - m1_ref[...]) * s1 + be1_ref[...]
    h = jnp.maximum(h, 0.0)
    h = jnp.dot(h, w2_ref[...], preferred_element_type=jnp.float32)
    h = h + b2_ref[...]
    deg = hist_ref[...][:, 0] + hist_ref[...][:, 1] + 1.0
    dinv = lax.rsqrt(deg)
    hs_ref[...] = h * dinv[:, None]


def _pre_call(x, w1, b1, g1, be1, m1, v1, w2, b2, hist_t):
    vec = pl.BlockSpec((1, D), lambda i: (0, 0))
    return pl.pallas_call(
        _pre_body,
        grid=(NBLK,),
        in_specs=[
            pl.BlockSpec((BR, D), lambda i: (i, 0)),
            pl.BlockSpec((D, D), lambda i: (0, 0)),
            vec, vec, vec, vec, vec,
            pl.BlockSpec((D, D), lambda i: (0, 0)),
            vec,
            pl.BlockSpec((BR, 2), lambda i: (i, 0)),
        ],
        out_specs=pl.BlockSpec((BR, D), lambda i: (i, 0)),
        out_shape=jax.ShapeDtypeStruct((N, D), jnp.float32),
    )(x, w1, b1, g1, be1, m1, v1, w2, b2, hist_t)


def _post_body(p0_ref, p1_ref, hs_ref, hist_ref, batch_ref,
               w3_ref, b3_ref, g2_ref, be2_ref, m2_ref, v2_ref,
               w4_ref, b4_ref, z_ref, pooled_acc, cnt_acc):
    i = pl.program_id(0)

    @pl.when(i == 0)
    def _():
        pooled_acc[...] = jnp.zeros((G, D), jnp.float32)
        cnt_acc[...] = jnp.zeros((G, D), jnp.float32)

    deg = hist_ref[...][:, 0] + hist_ref[...][:, 1] + 1.0
    dinv = lax.rsqrt(deg)
    inner = p0_ref[...] + p1_ref[...] + hs_ref[...]
    agg = jnp.maximum(inner * dinv[:, None], 0.0)

    b = batch_ref[...]
    p = (b == lax.broadcasted_iota(jnp.int32, (BR, G), 1)).astype(jnp.float32)
    dn = (((0,), (0,)), ((), ()))
    pooled_acc[...] += lax.dot_general(p, agg, dimension_numbers=dn,
                                       preferred_element_type=jnp.float32)
    cnt_acc[...] += lax.dot_general(p, jnp.ones((BR, D), jnp.float32),
                                    dimension_numbers=dn,
                                    preferred_element_type=jnp.float32)

    @pl.when(i == NBLK - 1)
    def _():
        pooled = pooled_acc[...] / jnp.maximum(cnt_acc[...], 1.0)
        z = jnp.dot(pooled, w3_ref[...], preferred_element_type=jnp.float32)
        z = z + b3_ref[...]
        s2 = g2_ref[...] * lax.rsqrt(v2_ref[...] + EPS)
        z = (z - m2_ref[...]) * s2 + be2_ref[...]
        z = jnp.maximum(z, 0.0)
        z_ref[...] = jnp.dot(z, w4_ref[...],
                             preferred_element_type=jnp.float32) + b4_ref[...]


def _post_call(p0, p1, hs, hist_t, batch2, w3, b3, g2, be2, m2, v2, w4, b4):
    vec = pl.BlockSpec((1, D), lambda i: (0, 0))
    node = pl.BlockSpec((BR, D), lambda i: (i, 0))
    return pl.pallas_call(
        _post_body,
        grid=(NBLK,),
        in_specs=[
            node, node, node,
            pl.BlockSpec((BR, 2), lambda i: (i, 0)),
            pl.BlockSpec((BR, 1), lambda i: (i, 0)),
            pl.BlockSpec((D, D), lambda i: (0, 0)),
            vec, vec, vec, vec, vec,
            pl.BlockSpec((D, 1), lambda i: (0, 0)),
            pl.BlockSpec((1, 1), lambda i: (0, 0)),
        ],
        out_specs=pl.BlockSpec((G, 1), lambda i: (0, 0)),
        out_shape=jax.ShapeDtypeStruct((G, 1), jnp.float32),
        scratch_shapes=[
            pltpu.VMEM((G, D), jnp.float32),
            pltpu.VMEM((G, D), jnp.float32),
        ],
    )(p0, p1, hs, hist_t, batch2, w3, b3, g2, be2, m2, v2, w4, b4)


def kernel(x, edge_index, edge_attr, batch, W1, b1, g1, be1, m1, v1,
           W2, b2, W3, b3, g2, be2, m2, v2, W4, b4):
    row = edge_index[0].astype(jnp.int32)
    col = edge_index[1].astype(jnp.int32)
    ea = edge_attr.astype(jnp.float32)

    hist = _hist_call(row, jnp.zeros((N, 1), jnp.float32),
                      jnp.ones((K, 1), jnp.float32))
    hist_t = jnp.transpose(hist[:, :, 0], (1, 0))

    hs = _pre_call(x, W1, b1.reshape(1, D), g1.reshape(1, D),
                   be1.reshape(1, D), m1.reshape(1, D), v1.reshape(1, D),
                   W2, b2.reshape(1, D), hist_t)

    parts = _scat_call(hs, row, col, ea,
                       jnp.zeros((N, D), jnp.float32))

    z = _post_call(parts[0], parts[1], hs, hist_t,
                   batch.astype(jnp.int32).reshape(N, 1),
                   W3, b3.reshape(1, D), g2.reshape(1, D), be2.reshape(1, D),
                   m2.reshape(1, D), v2.reshape(1, D), W4, b4.reshape(1, 1))
    return z


# speedup vs baseline: 11.3013x; 11.3013x over previous
import functools

import jax
import jax.numpy as jnp
from jax import lax
from jax.experimental import pallas as pl
from jax.experimental.pallas import tpu as pltpu
from jax.experimental.pallas import tpu_sc as plsc

N = 10000
E = 320000
D = 128
G = 64
EPS = 1e-5

NC = 2
NS = 16
NW = NC * NS
EPT = E // NW
K = 80
NWIN = EPT // K
NP = 10240
RPT = NP // NS

NBLK = 25
BR = N // NBLK

_mesh = plsc.VectorSubcoreMesh(core_axis_name="c", subcore_axis_name="s")


def _hist_body(row_hbm, zero_hbm, ones_hbm, out_hbm, idx_v, ones_v, hist_s):
    cid = lax.axis_index("c")
    sid = lax.axis_index("s")
    wid = sid * NC + cid

    pltpu.sync_copy(ones_hbm, ones_v)

    @pl.when(sid == 0)
    def _():
        pltpu.sync_copy(zero_hbm, hist_s)

    plsc.subcore_barrier()

    def body(w, carry):
        base = wid * EPT + w * K
        pltpu.sync_copy(row_hbm.at[pl.ds(base, K)], idx_v)
        pltpu.sync_copy(ones_v, hist_s.at[idx_v], add=True)
        return carry

    lax.fori_loop(0, NWIN, body, 0)
    plsc.subcore_barrier()

    @pl.when(sid == 0)
    def _():
        pltpu.sync_copy(hist_s, out_hbm.at[cid])


_hist_call = functools.partial(
    pl.kernel,
    out_type=jax.ShapeDtypeStruct((NC, N, 1), jnp.float32),
    mesh=_mesh,
    scratch_types=[
        pltpu.VMEM((K,), jnp.int32),
        pltpu.VMEM((K, 1), jnp.float32),
        pltpu.VMEM_SHARED((N, 1), jnp.float32),
    ],
)(_hist_body)


def _scat_body(hs_hbm, row_hbm, col_hbm, ea_hbm, zero_hbm, out_hbm,
               ridx_v, cidx_v, ea_v, rows_v, sem, acc_s):
    cid = lax.axis_index("c")
    sid = lax.axis_index("s")
    wid = sid * NC + cid

    pltpu.sync_copy(zero_hbm.at[pl.ds(sid * RPT, RPT)],
                    acc_s.at[pl.ds(sid * RPT, RPT)])
    plsc.subcore_barrier()

    def body(w, carry):
        base = wid * EPT + w * K
        pltpu.sync_copy(row_hbm.at[pl.ds(base, K)], ridx_v)
        pltpu.sync_copy(col_hbm.at[pl.ds(base, K)], cidx_v)
        pltpu.sync_copy(ea_hbm.at[pl.ds(base, K)], ea_v)
        pltpu.async_copy(hs_hbm.at[ridx_v], rows_v, sem).wait()

        def sbody(j, c2):
            s = jnp.zeros((16,), jnp.int32) + j
            sc = plsc.load_gather(ea_v, [s])
            for v in range(D // 16):
                sl = pl.ds(v * 16, 16)
                rows_v[j, sl] = rows_v[j, sl] * sc
            return c2

        lax.fori_loop(0, K, sbody, 0)
        pltpu.sync_copy(rows_v, acc_s.at[cidx_v], add=True)
        return carry

    lax.fori_loop(0, NWIN, body, 0)
    plsc.subcore_barrier()
    pltpu.sync_copy(acc_s.at[pl.ds(sid * RPT, RPT)],
                    out_hbm.at[cid, pl.ds(sid * RPT, RPT)])


_scat_call = functools.partial(
    pl.kernel,
    out_type=jax.ShapeDtypeStruct((NC, NP, D), jnp.float32),
    mesh=_mesh,
    scratch_types=[
        pltpu.VMEM((K,), jnp.int32),
        pltpu.VMEM((K,), jnp.int32),
        pltpu.VMEM((K,), jnp.float32),
        pltpu.VMEM((K, D), jnp.float32),
        pltpu.SemaphoreType.DMA,
        pltpu.VMEM_SHARED((NP, D), jnp.float32),
    ],
    compiler_params=pltpu.CompilerParams(needs_layout_passes=False),
)(_scat_body)


def _pre_body(x_ref, w1_ref, b1_ref, g1_ref, be1_ref, m1_ref, v1_ref,
              w2_ref, b2_ref, hist_ref, hs_ref):
    h = jnp.dot(x_ref[...], w1_ref[...], preferred_element_type=jnp.float32)
    h = h + b1_ref[...]
    s1 = g1_ref[...] * lax.rsqrt(v1_ref[...] + EPS)
    h = (h - m1_ref[...]) * s1 + be1_ref[...]
    h = jnp.maximum(h, 0.0)
    h = jnp.dot(h, w2_ref[...], preferred_element_type=jnp.float32)
    h = h + b2_ref[...]
    deg = hist_ref[...][:, 0] + hist_ref[...][:, 1] + 1.0
    dinv = lax.rsqrt(deg)
    hs_ref[...] = h * dinv[:, None]


def _pre_call(x, w1, b1, g1, be1, m1, v1, w2, b2, hist_t):
    vec = pl.BlockSpec((1, D), lambda i: (0, 0))
    return pl.pallas_call(
        _pre_body,
        grid=(NBLK,),
        in_specs=[
            pl.BlockSpec((BR, D), lambda i: (i, 0)),
            pl.BlockSpec((D, D), lambda i: (0, 0)),
            vec, vec, vec, vec, vec,
            pl.BlockSpec((D, D), lambda i: (0, 0)),
            vec,
            pl.BlockSpec((BR, 2), lambda i: (i, 0)),
        ],
        out_specs=pl.BlockSpec((BR, D), lambda i: (i, 0)),
        out_shape=jax.ShapeDtypeStruct((N, D), jnp.float32),
    )(x, w1, b1, g1, be1, m1, v1, w2, b2, hist_t)


def _post_body(p0_ref, p1_ref, hs_ref, hist_ref, batch_ref,
               w3_ref, b3_ref, g2_ref, be2_ref, m2_ref, v2_ref,
               w4_ref, b4_ref, z_ref, pooled_acc, cnt_acc):
    i = pl.program_id(0)

    @pl.when(i == 0)
    def _():
        pooled_acc[...] = jnp.zeros((G, D), jnp.float32)
        cnt_acc[...] = jnp.zeros((G, D), jnp.float32)

    deg = hist_ref[...][:, 0] + hist_ref[...][:, 1] + 1.0
    dinv = lax.rsqrt(deg)
    inner = p0_ref[...] + p1_ref[...] + hs_ref[...]
    agg = jnp.maximum(inner * dinv[:, None], 0.0)

    b = batch_ref[...]
    p = (b == lax.broadcasted_iota(jnp.int32, (BR, G), 1)).astype(jnp.float32)
    dn = (((0,), (0,)), ((), ()))
    pooled_acc[...] += lax.dot_general(p, agg, dimension_numbers=dn,
                                       preferred_element_type=jnp.float32)
    cnt_acc[...] += lax.dot_general(p, jnp.ones((BR, D), jnp.float32),
                                    dimension_numbers=dn,
                                    preferred_element_type=jnp.float32)

    @pl.when(i == NBLK - 1)
    def _():
        pooled = pooled_acc[...] / jnp.maximum(cnt_acc[...], 1.0)
        z = jnp.dot(pooled, w3_ref[...], preferred_element_type=jnp.float32)
        z = z + b3_ref[...]
        s2 = g2_ref[...] * lax.rsqrt(v2_ref[...] + EPS)
        z = (z - m2_ref[...]) * s2 + be2_ref[...]
        z = jnp.maximum(z, 0.0)
        z_ref[...] = jnp.dot(z, w4_ref[...],
                             preferred_element_type=jnp.float32) + b4_ref[...]


def _post_call(p0, p1, hs, hist_t, batch2, w3, b3, g2, be2, m2, v2, w4, b4):
    vec = pl.BlockSpec((1, D), lambda i: (0, 0))
    node = pl.BlockSpec((BR, D), lambda i: (i, 0))
    return pl.pallas_call(
        _post_body,
        grid=(NBLK,),
        in_specs=[
            node, node, node,
            pl.BlockSpec((BR, 2), lambda i: (i, 0)),
            pl.BlockSpec((BR, 1), lambda i: (i, 0)),
            pl.BlockSpec((D, D), lambda i: (0, 0)),
            vec, vec, vec, vec, vec,
            pl.BlockSpec((D, 1), lambda i: (0, 0)),
            pl.BlockSpec((1, 1), lambda i: (0, 0)),
        ],
        out_specs=pl.BlockSpec((G, 1), lambda i: (0, 0)),
        out_shape=jax.ShapeDtypeStruct((G, 1), jnp.float32),
        scratch_shapes=[
            pltpu.VMEM((G, D), jnp.float32),
            pltpu.VMEM((G, D), jnp.float32),
        ],
    )(p0, p1, hs, hist_t, batch2, w3, b3, g2, be2, m2, v2, w4, b4)


def kernel(x, edge_index, edge_attr, batch, W1, b1, g1, be1, m1, v1,
           W2, b2, W3, b3, g2, be2, m2, v2, W4, b4):
    row = edge_index[0].astype(jnp.int32)
    col = edge_index[1].astype(jnp.int32)
    ea = edge_attr.astype(jnp.float32)

    hist = _hist_call(row, jnp.zeros((N, 1), jnp.float32),
                      jnp.ones((K, 1), jnp.float32))
    hist_t = jnp.transpose(hist[:, :, 0], (1, 0))

    hs = _pre_call(x, W1, b1.reshape(1, D), g1.reshape(1, D),
                   be1.reshape(1, D), m1.reshape(1, D), v1.reshape(1, D),
                   W2, b2.reshape(1, D), hist_t)

    parts = _scat_call(hs, row, col, ea,
                       jnp.zeros((NP, D), jnp.float32))

    z = _post_call(parts[0, :N], parts[1, :N], hs, hist_t,
                   batch.astype(jnp.int32).reshape(N, 1),
                   W3, b3.reshape(1, D), g2.reshape(1, D), be2.reshape(1, D),
                   m2.reshape(1, D), v2.reshape(1, D), W4, b4.reshape(1, 1))
    return z


# speedup vs baseline: 17.8288x; 1.5776x over previous
import functools

import jax
import jax.numpy as jnp
from jax import lax
from jax.experimental import pallas as pl
from jax.experimental.pallas import tpu as pltpu
from jax.experimental.pallas import tpu_sc as plsc

N = 10000
E = 320000
D = 128
G = 64
EPS = 1e-5

NC = 2
NS = 16
NW = NC * NS
EPT = E // NW
K = 80
NWIN = EPT // K
NP = 10240
RPT = NP // NS

NBLK = 25
BR = N // NBLK

_mesh = plsc.VectorSubcoreMesh(core_axis_name="c", subcore_axis_name="s")


def _hist_body(row_hbm, zero_hbm, ones_hbm, out_hbm,
               i0, i1, i2, i3, ones_v, si0, si1, si2, si3, h0, h1, hist_s):
    cid = lax.axis_index("c")
    sid = lax.axis_index("s")
    wid = sid * NC + cid
    IDX = [i0, i1, i2, i3]
    SI = [si0, si1, si2, si3]
    HS = [h0, h1]

    def base_of(w):
        return wid * EPT + w * K

    def start_idx(w, q):
        pltpu.async_copy(row_hbm.at[pl.ds(base_of(w), K)], IDX[q], SI[q])

    def wait_idx(w, q):
        pltpu.make_async_copy(row_hbm.at[pl.ds(base_of(w), K)], IDX[q],
                              SI[q]).wait()

    def start_scat(q, e):
        pltpu.async_copy(ones_v, hist_s.at[IDX[q]], HS[e], add=True)

    def wait_scat(q, e):
        pltpu.make_async_copy(ones_v, hist_s.at[IDX[q]], HS[e]).wait()

    def _when(c, fn):
        if isinstance(c, bool):
            if c:
                fn()
        else:
            pl.when(c)(fn)

    pltpu.sync_copy(ones_hbm, ones_v)

    @pl.when(sid == 0)
    def _():
        pltpu.sync_copy(zero_hbm, hist_s)

    plsc.subcore_barrier()

    start_idx(0, 0)
    start_idx(1, 1)

    def step(w, k):
        wait_idx(w, k)
        _when(w >= 2, lambda: wait_scat((k - 2) % 4, k % 2))
        start_scat(k, k % 2)
        _when(w <= NWIN - 3, lambda: start_idx(w + 2, (k + 2) % 4))

    def quad(i, carry):
        for k in range(4):
            step(4 * i + k, k)
        return carry

    lax.fori_loop(0, (NWIN - 1) // 4, quad, 0)
    step(NWIN - 1, (NWIN - 1) % 4)
    wait_scat((NWIN - 2) % 4, (NWIN - 2) % 2)
    wait_scat((NWIN - 1) % 4, (NWIN - 1) % 2)
    plsc.subcore_barrier()

    @pl.when(sid == 0)
    def _():
        pltpu.sync_copy(hist_s, out_hbm.at[cid])


_hist_call = functools.partial(
    pl.kernel,
    out_type=jax.ShapeDtypeStruct((NC, N, 1), jnp.float32),
    mesh=_mesh,
    scratch_types=[
        pltpu.VMEM((K,), jnp.int32),
        pltpu.VMEM((K,), jnp.int32),
        pltpu.VMEM((K,), jnp.int32),
        pltpu.VMEM((K,), jnp.int32),
        pltpu.VMEM((K, 1), jnp.float32),
        pltpu.SemaphoreType.DMA,
        pltpu.SemaphoreType.DMA,
        pltpu.SemaphoreType.DMA,
        pltpu.SemaphoreType.DMA,
        pltpu.SemaphoreType.DMA,
        pltpu.SemaphoreType.DMA,
        pltpu.VMEM_SHARED((N, 1), jnp.float32),
    ],
)(_hist_body)


def _scat_body(hs_hbm, row_hbm, col_hbm, ea_hbm, zero_hbm, out_hbm,
               r0, r1, c0, c1, e0, e1, a0, a1,
               sr0, sr1, sc0, sc1, se0, se1, g0, g1, ss0, ss1, acc_s):
    cid = lax.axis_index("c")
    sid = lax.axis_index("s")
    wid = sid * NC + cid
    RIDX = [r0, r1]
    CIDX = [c0, c1]
    EA = [e0, e1]
    ROWS = [a0, a1]
    SR = [sr0, sr1]
    SC = [sc0, sc1]
    SE = [se0, se1]
    GS = [g0, g1]
    SS = [ss0, ss1]

    def base_of(w):
        return wid * EPT + w * K

    def start_ridx(w, k):
        pltpu.async_copy(row_hbm.at[pl.ds(base_of(w), K)], RIDX[k], SR[k])

    def wait_ridx(w, k):
        pltpu.make_async_copy(row_hbm.at[pl.ds(base_of(w), K)], RIDX[k],
                              SR[k]).wait()

    def start_cidx(w, k):
        pltpu.async_copy(col_hbm.at[pl.ds(base_of(w), K)], CIDX[k], SC[k])

    def wait_cidx(w, k):
        pltpu.make_async_copy(col_hbm.at[pl.ds(base_of(w), K)], CIDX[k],
                              SC[k]).wait()

    def start_ea(w, k):
        pltpu.async_copy(ea_hbm.at[pl.ds(base_of(w), K)], EA[k], SE[k])

    def wait_ea(w, k):
        pltpu.make_async_copy(ea_hbm.at[pl.ds(base_of(w), K)], EA[k],
                              SE[k]).wait()

    def start_gather(k):
        pltpu.async_copy(hs_hbm.at[RIDX[k]], ROWS[k], GS[k])

    def wait_gather(k):
        pltpu.make_async_copy(hs_hbm.at[RIDX[k]], ROWS[k], GS[k]).wait()

    def start_scatter(k):
        pltpu.async_copy(ROWS[k], acc_s.at[CIDX[k]], SS[k], add=True)

    def wait_scatter(k):
        pltpu.make_async_copy(ROWS[k], acc_s.at[CIDX[k]], SS[k]).wait()

    def scale(k):
        rows = ROWS[k]
        ea = EA[k]

        def sbody(j, c2):
            sc_ = plsc.load_gather(ea, [jnp.zeros((16,), jnp.int32) + j])
            for v in range(D // 16):
                sl = pl.ds(v * 16, 16)
                rows[j, sl] = rows[j, sl] * sc_
            return c2

        lax.fori_loop(0, K, sbody, 0, unroll=4)

    def _when(c, fn):
        if isinstance(c, bool):
            if c:
                fn()
        else:
            pl.when(c)(fn)

    pltpu.sync_copy(zero_hbm.at[pl.ds(sid * RPT, RPT)],
                    acc_s.at[pl.ds(sid * RPT, RPT)])
    plsc.subcore_barrier()

    start_ridx(0, 0)
    start_ea(0, 0)
    start_cidx(0, 0)
    start_ridx(1, 1)
    start_ea(1, 1)
    wait_ridx(0, 0)
    start_gather(0)
    wait_ea(0, 0)

    def step(w, k):
        wait_gather(k)
        scale(k)
        wait_cidx(w, k)
        start_scatter(k)
        _when(w >= 1, lambda: wait_scatter(1 - k))

        def _next():
            start_cidx(w + 1, 1 - k)
            wait_ridx(w + 1, 1 - k)
            wait_ea(w + 1, 1 - k)
            start_gather(1 - k)

        _when(w <= NWIN - 2, _next)

        def _next2():
            start_ridx(w + 2, k)
            start_ea(w + 2, k)

        _when(w <= NWIN - 3, _next2)

    def pair(i, carry):
        step(2 * i, 0)
        step(2 * i + 1, 1)
        return carry

    lax.fori_loop(0, (NWIN - 1) // 2, pair, 0)
    step(NWIN - 1, (NWIN - 1) % 2)
    wait_scatter((NWIN - 1) % 2)
    plsc.subcore_barrier()
    pltpu.sync_copy(acc_s.at[pl.ds(sid * RPT, RPT)],
                    out_hbm.at[cid, pl.ds(sid * RPT, RPT)])


_scat_call = functools.partial(
    pl.kernel,
    out_type=jax.ShapeDtypeStruct((NC, NP, D), jnp.float32),
    mesh=_mesh,
    scratch_types=[
        pltpu.VMEM((K,), jnp.int32),
        pltpu.VMEM((K,), jnp.int32),
        pltpu.VMEM((K,), jnp.int32),
        pltpu.VMEM((K,), jnp.int32),
        pltpu.VMEM((K,), jnp.float32),
        pltpu.VMEM((K,), jnp.float32),
        pltpu.VMEM((K, D), jnp.float32),
        pltpu.VMEM((K, D), jnp.float32),
        pltpu.SemaphoreType.DMA,
        pltpu.SemaphoreType.DMA,
        pltpu.SemaphoreType.DMA,
        pltpu.SemaphoreType.DMA,
        pltpu.SemaphoreType.DMA,
        pltpu.SemaphoreType.DMA,
        pltpu.SemaphoreType.DMA,
        pltpu.SemaphoreType.DMA,
        pltpu.SemaphoreType.DMA,
        pltpu.SemaphoreType.DMA,
        pltpu.VMEM_SHARED((NP, D), jnp.float32),
    ],
    compiler_params=pltpu.CompilerParams(needs_layout_passes=False),
)(_scat_body)


def _pre_body(x_ref, w1_ref, b1_ref, g1_ref, be1_ref, m1_ref, v1_ref,
              w2_ref, b2_ref, hist_ref, hs_ref):
    h = jnp.dot(x_ref[...], w1_ref[...], preferred_element_type=jnp.float32)
    h = h + b1_ref[...]
    s1 = g1_ref[...] * lax.rsqrt(v1_ref[...] + EPS)
    h = (h - m1_ref[...]) * s1 + be1_ref[...]
    h = jnp.maximum(h, 0.0)
    h = jnp.dot(h, w2_ref[...], preferred_element_type=jnp.float32)
    h = h + b2_ref[...]
    deg = hist_ref[...][:, 0] + hist_ref[...][:, 1] + 1.0
    dinv = lax.rsqrt(deg)
    hs_ref[...] = h * dinv[:, None]


def _pre_call(x, w1, b1, g1, be1, m1, v1, w2, b2, hist_t):
    vec = pl.BlockSpec((1, D), lambda i: (0, 0))
    return pl.pallas_call(
        _pre_body,
        grid=(NBLK,),
        in_specs=[
            pl.BlockSpec((BR, D), lambda i: (i, 0)),
            pl.BlockSpec((D, D), lambda i: (0, 0)),
            vec, vec, vec, vec, vec,
            pl.BlockSpec((D, D), lambda i: (0, 0)),
            vec,
            pl.BlockSpec((BR, 2), lambda i: (i, 0)),
        ],
        out_specs=pl.BlockSpec((BR, D), lambda i: (i, 0)),
        out_shape=jax.ShapeDtypeStruct((N, D), jnp.float32),
    )(x, w1, b1, g1, be1, m1, v1, w2, b2, hist_t)


def _post_body(p0_ref, p1_ref, hs_ref, hist_ref, batch_ref,
               w3_ref, b3_ref, g2_ref, be2_ref, m2_ref, v2_ref,
               w4_ref, b4_ref, z_ref, pooled_acc, cnt_acc):
    i = pl.program_id(0)

    @pl.when(i == 0)
    def _():
        pooled_acc[...] = jnp.zeros((G, D), jnp.float32)
        cnt_acc[...] = jnp.zeros((G, D), jnp.float32)

    deg = hist_ref[...][:, 0] + hist_ref[...][:, 1] + 1.0
    dinv = lax.rsqrt(deg)
    inner = p0_ref[...] + p1_ref[...] + hs_ref[...]
    agg = jnp.maximum(inner * dinv[:, None], 0.0)

    b = batch_ref[...]
    p = (b == lax.broadcasted_iota(jnp.int32, (BR, G), 1)).astype(jnp.float32)
    dn = (((0,), (0,)), ((), ()))
    pooled_acc[...] += lax.dot_general(p, agg, dimension_numbers=dn,
                                       preferred_element_type=jnp.float32)
    cnt_acc[...] += lax.dot_general(p, jnp.ones((BR, D), jnp.float32),
                                    dimension_numbers=dn,
                                    preferred_element_type=jnp.float32)

    @pl.when(i == NBLK - 1)
    def _():
        pooled = pooled_acc[...] / jnp.maximum(cnt_acc[...], 1.0)
        z = jnp.dot(pooled, w3_ref[...], preferred_element_type=jnp.float32)
        z = z + b3_ref[...]
        s2 = g2_ref[...] * lax.rsqrt(v2_ref[...] + EPS)
        z = (z - m2_ref[...]) * s2 + be2_ref[...]
        z = jnp.maximum(z, 0.0)
        z_ref[...] = jnp.dot(z, w4_ref[...],
                             preferred_element_type=jnp.float32) + b4_ref[...]


def _post_call(p0, p1, hs, hist_t, batch2, w3, b3, g2, be2, m2, v2, w4, b4):
    vec = pl.BlockSpec((1, D), lambda i: (0, 0))
    node = pl.BlockSpec((BR, D), lambda i: (i, 0))
    return pl.pallas_call(
        _post_body,
        grid=(NBLK,),
        in_specs=[
            node, node, node,
            pl.BlockSpec((BR, 2), lambda i: (i, 0)),
            pl.BlockSpec((BR, 1), lambda i: (i, 0)),
            pl.BlockSpec((D, D), lambda i: (0, 0)),
            vec, vec, vec, vec, vec,
            pl.BlockSpec((D, 1), lambda i: (0, 0)),
            pl.BlockSpec((1, 1), lambda i: (0, 0)),
        ],
        out_specs=pl.BlockSpec((G, 1), lambda i: (0, 0)),
        out_shape=jax.ShapeDtypeStruct((G, 1), jnp.float32),
        scratch_shapes=[
            pltpu.VMEM((G, D), jnp.float32),
            pltpu.VMEM((G, D), jnp.float32),
        ],
    )(p0, p1, hs, hist_t, batch2, w3, b3, g2, be2, m2, v2, w4, b4)


def kernel(x, edge_index, edge_attr, batch, W1, b1, g1, be1, m1, v1,
           W2, b2, W3, b3, g2, be2, m2, v2, W4, b4):
    row = edge_index[0].astype(jnp.int32)
    col = edge_index[1].astype(jnp.int32)
    ea = edge_attr.astype(jnp.float32)

    hist = _hist_call(row, jnp.zeros((N, 1), jnp.float32),
                      jnp.ones((K, 1), jnp.float32))
    hist_t = jnp.transpose(hist[:, :, 0], (1, 0))

    hs = _pre_call(x, W1, b1.reshape(1, D), g1.reshape(1, D),
                   be1.reshape(1, D), m1.reshape(1, D), v1.reshape(1, D),
                   W2, b2.reshape(1, D), hist_t)

    parts = _scat_call(hs, row, col, ea,
                       jnp.zeros((NP, D), jnp.float32))

    z = _post_call(parts[0, :N], parts[1, :N], hs, hist_t,
                   batch.astype(jnp.int32).reshape(N, 1),
                   W3, b3.reshape(1, D), g2.reshape(1, D), be2.reshape(1, D),
                   m2.reshape(1, D), v2.reshape(1, D), W4, b4.reshape(1, 1))
    return z
